# Initial kernel scaffold; baseline (speedup 1.0000x reference)
import jax
import jax.numpy as jnp
from jax.experimental import pallas as pl


def kernel(x, edge_index, edge_weight, batch, batch_size, W0, W1, b_cheb, W_ih, W_hh, b_ih, b_hh, W_fc, b_fc):
    raise NotImplementedError("write your pallas kernel here")


# speedup vs baseline: 15.7424x; 15.7424x over previous
import functools

import jax
import jax.numpy as jnp
from jax import lax
from jax.experimental import pallas as pl
from jax.experimental.pallas import tpu as pltpu
from jax.experimental.pallas import tpu_sc as plsc

NC = 2
NS = 16
B_GRAPHS = 64

F32 = jnp.float32
I32 = jnp.int32


def _divisor_le(n, cap):
    for d in range(cap, 0, -1):
        if n % d == 0:
            return d
    return 1


def _make_deg_kernel(n_rows, N_pad, KG=8):
    rows_per_w = n_rows // (NC * NS)
    G = rows_per_w // KG
    npt = N_pad // NS
    mesh = plsc.VectorSubcoreMesh(core_axis_name="c", subcore_axis_name="s",
                                  num_cores=NC, num_subcores=NS)

    @functools.partial(
        pl.kernel,
        out_type=jax.ShapeDtypeStruct((NC * N_pad,), F32),
        mesh=mesh,
        scratch_types=[
            pltpu.VMEM((KG, 128), I32),
            pltpu.VMEM((KG, 128), F32),
            pltpu.VMEM((npt,), F32),
            pltpu.VMEM_SHARED((N_pad,), F32),
        ],
    )
    def deg_kernel(row_hbm, ew_hbm, out_hbm, idx_v, w_v, z_v, deg_sh):
        c = lax.axis_index("c")
        s = lax.axis_index("s")

        def zb(i, carry):
            z_v[pl.ds(i * 16, 16)] = jnp.zeros((16,), F32)
            return carry
        lax.fori_loop(0, npt // 16, zb, 0)
        pltpu.sync_copy(z_v, deg_sh.at[pl.ds(s * npt, npt)])
        plsc.subcore_barrier()

        wid = s * NC + c
        base = wid * rows_per_w

        def gb(g, carry):
            r0 = base + g * KG
            pltpu.sync_copy(row_hbm.at[pl.ds(r0, KG)], idx_v)
            pltpu.sync_copy(ew_hbm.at[pl.ds(r0, KG)], w_v)
            for j in range(KG):
                pltpu.sync_copy(w_v.at[j], deg_sh.at[idx_v.at[j]], add=True)
            return carry
        lax.fori_loop(0, G, gb, 0)

        plsc.subcore_barrier()
        pltpu.sync_copy(deg_sh.at[pl.ds(s * npt, npt)], z_v)
        pltpu.sync_copy(z_v, out_hbm.at[pl.ds(c * N_pad + s * npt, npt)])

    return deg_kernel


def _make_agg_kernel(n_rows, N_pad, D, KG=4):
    rows_per_t = n_rows // NS
    G = rows_per_t // KG
    npt = N_pad // NS
    ZR = _divisor_le(npt, 128)
    mesh = plsc.VectorSubcoreMesh(core_axis_name="c", subcore_axis_name="s",
                                  num_cores=NC, num_subcores=NS)

    @functools.partial(
        pl.kernel,
        out_type=jax.ShapeDtypeStruct((NC, N_pad, D), F32),
        mesh=mesh,
        scratch_types=[
            pltpu.VMEM((KG, 128), I32),
            pltpu.VMEM((KG, 128), I32),
            pltpu.VMEM((KG, 128), F32),
            pltpu.VMEM((KG, 128, D), F32),
            pltpu.VMEM((ZR, D), F32),
            pltpu.VMEM_SHARED((N_pad, D), F32),
            pltpu.SemaphoreType.DMA,
        ] + [pltpu.VMEM((128,), I32) for _ in range(2 * KG)],
        compiler_params=pltpu.CompilerParams(use_tc_tiling_on_sc=False),
    )
    def agg_kernel(row_hbm, col_hbm, ew_hbm, tbl_hbm, out_hbm,
                   idx_v, col_v, w_v, rows_v, z_v, agg_sh, sem, *ibufs):
        gbufs, sbufs = ibufs[:KG], ibufs[KG:]
        c = lax.axis_index("c")
        s = lax.axis_index("s")

        z16 = jnp.zeros((16,), F32)

        def zb(i, carry):
            for f0 in range(0, D, 16):
                z_v[i, f0:f0 + 16] = z16
            return carry
        lax.fori_loop(0, ZR, zb, 0)

        def zc(i, carry):
            pltpu.sync_copy(z_v, agg_sh.at[pl.ds(s * npt + i * ZR, ZR)])
            return carry
        lax.fori_loop(0, npt // ZR, zc, 0)
        plsc.subcore_barrier()

        base = s * rows_per_t
        coff = c * N_pad

        def gb(g, carry):
            r0 = base + g * KG
            pltpu.sync_copy(row_hbm.at[pl.ds(r0, KG)], idx_v)
            pltpu.sync_copy(col_hbm.at[pl.ds(r0, KG)], col_v)
            pltpu.sync_copy(ew_hbm.at[pl.ds(r0, KG)], w_v)

            for j in range(KG):
                def tb(i, carry2):
                    gbufs[j][pl.ds(i * 16, 16)] = (
                        idx_v[j, pl.ds(i * 16, 16)] + coff)
                    sbufs[j][pl.ds(i * 16, 16)] = col_v[j, pl.ds(i * 16, 16)]
                    return carry2
                lax.fori_loop(0, 8, tb, 0)

            descs = [pltpu.async_copy(tbl_hbm.at[gbufs[j]], rows_v.at[j], sem)
                     for j in range(KG)]
            for d in descs:
                d.wait()

            for j in range(KG):
                def sb(i, carry2):
                    w16 = w_v[j, pl.ds(i * 16, 16)]
                    for l in range(16):
                        e = i * 16 + l
                        w = lax.broadcast_in_dim(w16[l], (16,), ())
                        for f0 in range(0, D, 16):
                            rows_v[j, e, f0:f0 + 16] = (
                                rows_v[j, e, f0:f0 + 16] * w)
                    return carry2
                lax.fori_loop(0, 8, sb, 0)

            for j in range(KG):
                pltpu.sync_copy(rows_v.at[j], agg_sh.at[sbufs[j]], add=True)
            return carry
        lax.fori_loop(0, G, gb, 0)

        plsc.subcore_barrier()
        def oc(i, carry):
            pltpu.sync_copy(agg_sh.at[pl.ds(s * npt + i * ZR, ZR)], z_v)
            pltpu.sync_copy(z_v, out_hbm.at[c, pl.ds(s * npt + i * ZR, ZR)])
            return carry
        lax.fori_loop(0, npt // ZR, oc, 0)

    return agg_kernel


def _make_tc1(N_pad, R, D_IN, D_HID, DP):
    NB = N_pad // R
    D = D_HID // 2

    def body(x_ref, degp_ref, w0_ref, w1_ref, xw0_ref, tbl_ref):
        x = x_ref[...]
        deg = degp_ref[0] + degp_ref[1]
        dis = jnp.where(deg > 0.0,
                        lax.rsqrt(jnp.maximum(deg, 1e-30)), 0.0)
        xw0_ref[...] = jnp.dot(x, w0_ref[...], preferred_element_type=F32)
        t = jnp.dot(x * dis, w1_ref[...], preferred_element_type=F32)
        zpad = jnp.zeros((R, DP - D), F32)
        tbl_ref[0] = jnp.concatenate([t[:, :D], zpad], axis=1)
        tbl_ref[1] = jnp.concatenate([t[:, D:], zpad], axis=1)

    return pl.pallas_call(
        body,
        grid=(NB,),
        in_specs=[
            pl.BlockSpec((R, D_IN), lambda i: (i, 0)),
            pl.BlockSpec((2, R, 1), lambda i: (0, i, 0)),
            pl.BlockSpec((D_IN, D_HID), lambda i: (0, 0)),
            pl.BlockSpec((D_IN, D_HID), lambda i: (0, 0)),
        ],
        out_specs=[
            pl.BlockSpec((R, D_HID), lambda i: (i, 0)),
            pl.BlockSpec((2, R, DP), lambda i: (0, i, 0)),
        ],
        out_shape=[
            jax.ShapeDtypeStruct((N_pad, D_HID), F32),
            jax.ShapeDtypeStruct((2, N_pad, DP), F32),
        ],
    )


def _make_tc2(N_pad, R, D_HID, D_RNN, N_CLS, DP):
    NB = N_pad // R
    D = D_HID // 2

    def body(xw0_ref, agg_ref, degp_ref, bch_ref, batch_ref,
             wih_ref, whh_ref, bih_ref, bhh_ref, wfc_ref, bfc_ref,
             out_ref, pooled_acc, pre_scr, outs_scr):
        i = pl.program_id(0)

        @pl.when(i == 0)
        def _init():
            pooled_acc[...] = jnp.zeros_like(pooled_acc)

        deg = degp_ref[0] + degp_ref[1]
        dis = jnp.where(deg > 0.0,
                        lax.rsqrt(jnp.maximum(deg, 1e-30)), 0.0)
        agg = jnp.concatenate([agg_ref[0][:, :D], agg_ref[1][:, :D]],
                              axis=1)
        h = xw0_ref[...] + bch_ref[...] - dis * agg
        h = jnp.maximum(h, 0.0)

        bids = batch_ref[...]
        onehot = (bids == lax.broadcasted_iota(I32, (R, B_GRAPHS), 1)
                  ).astype(F32)
        pooled_acc[...] += lax.dot_general(
            onehot, h, (((0,), (0,)), ((), ())), preferred_element_type=F32)

        @pl.when(i == NB - 1)
        def _tail():
            pooled = pooled_acc[...]
            pre = lax.dot_general(
                pooled, wih_ref[...], (((1,), (1,)), ((), ())),
                preferred_element_type=F32) + bih_ref[...] + bhh_ref[...]
            pre_scr[...] = pre

            def step(t, hprev):
                z = pre_scr[pl.ds(t, 1), :] + lax.dot_general(
                    hprev, whh_ref[...], (((1,), (1,)), ((), ())),
                    preferred_element_type=F32)
                hn = jnp.tanh(z)
                outs_scr[pl.ds(t, 1), :] = hn
                return hn
            lax.fori_loop(0, B_GRAPHS, step, jnp.zeros((1, D_RNN), F32))

            logits = lax.dot_general(
                outs_scr[...], wfc_ref[...], (((1,), (1,)), ((), ())),
                preferred_element_type=F32) + bfc_ref[...]
            m = jnp.max(logits, axis=1, keepdims=True)
            sh = logits - m
            out_ref[...] = sh - jnp.log(
                jnp.sum(jnp.exp(sh), axis=1, keepdims=True))

    return pl.pallas_call(
        body,
        grid=(NB,),
        in_specs=[
            pl.BlockSpec((R, D_HID), lambda i: (i, 0)),
            pl.BlockSpec((2, R, DP), lambda i: (0, i, 0)),
            pl.BlockSpec((2, R, 1), lambda i: (0, i, 0)),
            pl.BlockSpec((1, D_HID), lambda i: (0, 0)),
            pl.BlockSpec((R, 1), lambda i: (i, 0)),
            pl.BlockSpec((D_RNN, D_HID), lambda i: (0, 0)),
            pl.BlockSpec((D_RNN, D_RNN), lambda i: (0, 0)),
            pl.BlockSpec((1, D_RNN), lambda i: (0, 0)),
            pl.BlockSpec((1, D_RNN), lambda i: (0, 0)),
            pl.BlockSpec((N_CLS, D_RNN), lambda i: (0, 0)),
            pl.BlockSpec((1, N_CLS), lambda i: (0, 0)),
        ],
        out_specs=pl.BlockSpec((B_GRAPHS, N_CLS), lambda i: (0, 0)),
        out_shape=jax.ShapeDtypeStruct((B_GRAPHS, N_CLS), F32),
        scratch_shapes=[
            pltpu.VMEM((B_GRAPHS, D_HID), F32),
            pltpu.VMEM((B_GRAPHS, D_RNN), F32),
            pltpu.VMEM((B_GRAPHS, D_RNN), F32),
        ],
    )


def kernel(x, edge_index, edge_weight, batch, batch_size,
           W0, W1, b_cheb, W_ih, W_hh, b_ih, b_hh, W_fc, b_fc):
    N, D_IN = x.shape
    E = edge_index.shape[1]
    D_HID = W0.shape[1]
    D_RNN = W_ih.shape[0]
    N_CLS = W_fc.shape[0]
    DP = 32

    R = 1024
    N_pad = -(-N // R) * R
    n_rows = -(-E // (128 * NC * NS * 8)) * (NC * NS * 8)
    E_pad = n_rows * 128

    row = edge_index[0].astype(I32)
    col = edge_index[1].astype(I32)
    pad_e = E_pad - E
    row_p = jnp.pad(row, (0, pad_e)).reshape(n_rows, 128)
    col_p = jnp.pad(col, (0, pad_e)).reshape(n_rows, 128)
    ew_p = jnp.pad(edge_weight, (0, pad_e)).reshape(n_rows, 128)

    degp = _make_deg_kernel(n_rows, N_pad)(row_p, ew_p)
    degp3 = degp.reshape(NC, N_pad, 1)

    x_p = jnp.pad(x, ((0, N_pad - N), (0, 0)))
    xw0, tbl = _make_tc1(N_pad, R, D_IN, D_HID, DP)(x_p, degp3, W0, W1)
    tbl2 = tbl.reshape(NC * N_pad, DP)

    aggs = _make_agg_kernel(n_rows, N_pad, DP)(row_p, col_p, ew_p, tbl2)

    batch_p = jnp.pad(batch.astype(I32), (0, N_pad - N),
                      constant_values=B_GRAPHS).reshape(N_pad, 1)
    out = _make_tc2(N_pad, R, D_HID, D_RNN, N_CLS, DP)(
        xw0, aggs, degp3, b_cheb.reshape(1, D_HID), batch_p,
        W_ih, W_hh, b_ih.reshape(1, D_RNN), b_hh.reshape(1, D_RNN),
        W_fc, b_fc.reshape(1, N_CLS))
    return out
